# baseline (speedup 1.0000x reference)
import functools

import jax
import jax.numpy as jnp
from jax import lax
from jax.experimental import pallas as pl
from jax.experimental.pallas import tpu as pltpu
from jax.experimental.pallas import tpu_sc as plsc

NUM_CLASSES = 512
EMB_DIM = 512
N_SUPPORT = 16384
N_QUERY = 8192

_NC = 2
_NS = 16
_NW = _NC * _NS
_RPW = N_SUPPORT // _NW
_CHUNK = 64
_NCHUNKS = _RPW // _CHUNK
_NV = EMB_DIM // 16


def _sc_segment_sum_body(support_hbm, labels_hbm, nextlo_hbm,
                         dense_hbm, side_hbm,
                         rows_a, rows_b, flush_v, zeros_v, labs_s, nextlo_s,
                         sem_a, sem_b):
    cid = lax.axis_index("c")
    sid = lax.axis_index("s")
    w = cid * _NS + sid
    row0 = w * _RPW

    pltpu.sync_copy(labels_hbm.at[pl.ds(row0, _RPW)],
                    labs_s.at[pl.ds(0, _RPW)])
    pltpu.sync_copy(nextlo_hbm, nextlo_s.at[pl.ds(0, _NW)])

    def _lab_at(i):
        return labs_s[pl.ds(i, 16)][0]

    for j in range(_NV):
        zeros_v[pl.ds(j * 16, 16)] = jnp.zeros((16,), jnp.float32)

    lo = _lab_at(0)
    nextlo = nextlo_s[pl.ds(w, 16)][0]

    def _zero_dense(c, _):
        pltpu.sync_copy(zeros_v, dense_hbm.at[pl.ds(c * EMB_DIM, EMB_DIM)])
        return 0

    @pl.when(w == 0)
    def _():
        lax.fori_loop(0, lo, _zero_dense, 0)

    def _flush(cls, acc):
        for j in range(_NV):
            flush_v[pl.ds(j * 16, 16)] = acc[j]

        @pl.when(cls == lo)
        def _():
            pltpu.sync_copy(flush_v,
                            side_hbm.at[pl.ds(2 * w * EMB_DIM, EMB_DIM)])

        @pl.when(cls != lo)
        def _():
            pltpu.sync_copy(flush_v,
                            dense_hbm.at[pl.ds(cls * EMB_DIM, EMB_DIM)])

    zero16 = jnp.zeros((16,), jnp.float32)

    def _gather_start(k, buf, sem):
        pltpu.async_copy(
            support_hbm.at[pl.ds((row0 + k * _CHUNK) * EMB_DIM,
                                 _CHUNK * EMB_DIM)], buf, sem)

    def _gather_wait(buf, sem):
        pltpu.make_async_copy(
            support_hbm.at[pl.ds(0, _CHUNK * EMB_DIM)], buf, sem).wait()

    def _chunk_walk(buf, k, carry):
        def _row(r, carry):
            cur = carry[0]
            acc = carry[1:]
            lab = _lab_at(k * _CHUNK + r)
            changed = lab != cur

            @pl.when(changed)
            def _():
                _flush(cur, acc)
                lax.fori_loop(cur + 1, lab, _zero_dense, 0)

            row = [buf[pl.ds(r * EMB_DIM + j * 16, 16)]
                   for j in range(_NV)]
            new_acc = [
                jnp.where(changed, row[j], acc[j] + row[j])
                for j in range(_NV)
            ]
            return (lab,) + tuple(new_acc)

        return lax.fori_loop(0, _CHUNK, _row, carry)

    _gather_start(0, rows_a, sem_a)

    def _outer(t, carry):
        k0 = 2 * t
        _gather_wait(rows_a, sem_a)
        _gather_start(k0 + 1, rows_b, sem_b)
        carry = _chunk_walk(rows_a, k0, carry)
        _gather_wait(rows_b, sem_b)

        @pl.when(t < _NCHUNKS // 2 - 1)
        def _():
            _gather_start(k0 + 2, rows_a, sem_a)

        return _chunk_walk(rows_b, k0 + 1, carry)

    init = (lo,) + tuple(zero16 for _ in range(_NV))
    final = lax.fori_loop(0, _NCHUNKS // 2, _outer, init)
    hi = final[0]
    acc = final[1:]

    _flush(hi, acc)

    @pl.when(hi == lo)
    def _():
        pltpu.sync_copy(zeros_v,
                        side_hbm.at[pl.ds((2 * w + 1) * EMB_DIM, EMB_DIM)])

    lax.fori_loop(hi, nextlo + 1, _zero_dense, 0)


@functools.lru_cache(maxsize=1)
def _build_sc_segment_sum():
    return functools.partial(
        pl.kernel,
        out_type=(
            jax.ShapeDtypeStruct((NUM_CLASSES * EMB_DIM,), jnp.float32),
            jax.ShapeDtypeStruct((2 * _NW * EMB_DIM,), jnp.float32),
        ),
        mesh=plsc.VectorSubcoreMesh(core_axis_name="c", subcore_axis_name="s",
                                    num_cores=_NC, num_subcores=_NS),
        scratch_types=[
            pltpu.VMEM((_CHUNK * EMB_DIM,), jnp.float32),
            pltpu.VMEM((_CHUNK * EMB_DIM,), jnp.float32),
            pltpu.VMEM((EMB_DIM,), jnp.float32),
            pltpu.VMEM((EMB_DIM,), jnp.float32),
            pltpu.VMEM((_RPW + 16,), jnp.int32),
            pltpu.VMEM((_NW + 16,), jnp.int32),
            pltpu.SemaphoreType.DMA,
            pltpu.SemaphoreType.DMA,
        ],
    )(_sc_segment_sum_body)


_QBLK = 256
_LABCHUNK = 2048


def _tc_enc_body(q_ref, w_ref, b_ref, qe_ref):
    qe_ref[...] = lax.dot_general(
        q_ref[...], w_ref[...], (((1,), (1,)), ((), ())),
        preferred_element_type=jnp.float32,
        precision=lax.Precision.DEFAULT) + b_ref[...]


_tc_enc = pl.pallas_call(
    _tc_enc_body,
    grid=(N_QUERY // _QBLK,),
    in_specs=[
        pl.BlockSpec((_QBLK, EMB_DIM), lambda i: (i, 0)),
        pl.BlockSpec((EMB_DIM, EMB_DIM), lambda i: (0, 0)),
        pl.BlockSpec((1, EMB_DIM), lambda i: (0, 0)),
    ],
    out_specs=pl.BlockSpec((_QBLK, EMB_DIM), lambda i: (i, 0)),
    out_shape=jax.ShapeDtypeStruct((N_QUERY, EMB_DIM), jnp.float32),
)


def _tc_main_body(qe_in_ref, w_ref, b_ref, dense_ref, side_ref, bidx_ref,
                  lab_ref, out_ref, protos_ref, p2_ref):
    i = pl.program_id(0)

    @pl.when(i == 0)
    def _():
        cls = lax.broadcasted_iota(jnp.int32, (NUM_CLASSES, 1), 0)
        clsf = cls.astype(jnp.float32)
        onehot = (clsf == bidx_ref[...]).astype(jnp.float32)
        sums = dense_ref[...] + lax.dot_general(
            onehot, side_ref[...], (((1,), (0,)), ((), ())),
            preferred_element_type=jnp.float32)
        acc = jnp.zeros((NUM_CLASSES, 1), jnp.float32)
        labs = lab_ref[...]
        for r in range(N_SUPPORT // _LABCHUNK):
            row = labs[r:r + 1, :]
            acc = acc + jnp.sum((row == cls).astype(jnp.float32), axis=1,
                                keepdims=True)
        inv = 1.0 / jnp.maximum(acc, 1.0)
        pm = lax.dot_general(sums, w_ref[...], (((1,), (1,)), ((), ())),
                             preferred_element_type=jnp.float32)
        protos = pm * inv + (acc * inv) * b_ref[...]
        protos_ref[...] = protos
        ones = jnp.ones((1, EMB_DIM), jnp.float32)
        p2_ref[...] = lax.dot_general(ones, protos * protos,
                                      (((1,), (1,)), ((), ())),
                                      preferred_element_type=jnp.float32)

    qe = qe_in_ref[...]
    protos = protos_ref[...]
    cross = lax.dot_general(qe, protos, (((1,), (1,)), ((), ())),
                            preferred_element_type=jnp.float32,
                            precision=lax.Precision.DEFAULT)
    q2 = jnp.sum(qe * qe, axis=1, keepdims=True)
    d2 = q2 + p2_ref[0:1, :] - 2.0 * cross
    out_ref[...] = -jnp.sqrt(jnp.maximum(d2, 1e-12))


_tc_main = pl.pallas_call(
    _tc_main_body,
    grid=(N_QUERY // _QBLK,),
    in_specs=[
        pl.BlockSpec((_QBLK, EMB_DIM), lambda i: (i, 0)),
        pl.BlockSpec((EMB_DIM, EMB_DIM), lambda i: (0, 0)),
        pl.BlockSpec((1, EMB_DIM), lambda i: (0, 0)),
        pl.BlockSpec((NUM_CLASSES, EMB_DIM), lambda i: (0, 0)),
        pl.BlockSpec((2 * _NW, EMB_DIM), lambda i: (0, 0)),
        pl.BlockSpec((1, 2 * _NW), lambda i: (0, 0)),
        pl.BlockSpec((N_SUPPORT // _LABCHUNK, _LABCHUNK), lambda i: (0, 0)),
    ],
    out_specs=pl.BlockSpec((_QBLK, NUM_CLASSES), lambda i: (i, 0)),
    out_shape=jax.ShapeDtypeStruct((N_QUERY, NUM_CLASSES), jnp.float32),
    scratch_shapes=[
        pltpu.VMEM((NUM_CLASSES, EMB_DIM), jnp.float32),
        pltpu.VMEM((1, NUM_CLASSES), jnp.float32),
    ],
)


def kernel(query_set, support_set, support_labels, W, b):
    labels = support_labels.astype(jnp.int32)
    nextlo = jnp.concatenate(
        [labels[_RPW::_RPW],
         jnp.full((1,), NUM_CLASSES, jnp.int32)])
    dense, side = _build_sc_segment_sum()(support_set.reshape(-1), labels,
                                          nextlo)
    dense = dense.reshape(NUM_CLASSES, EMB_DIM)
    side = side.reshape(2 * _NW, EMB_DIM)
    heads = labels[::_RPW]
    tails = labels[_RPW - 1::_RPW]
    bidx = jnp.stack([heads, tails], axis=1).reshape(1, 2 * _NW)
    bidx = bidx.astype(jnp.float32)
    b2 = b.reshape(1, EMB_DIM)
    lab2d = labels.reshape(N_SUPPORT // _LABCHUNK, _LABCHUNK)
    qe = _tc_enc(query_set, W, b2)
    return _tc_main(qe, W, b2, dense, side, bidx, lab2d)


# speedup vs baseline: 1.0045x; 1.0045x over previous
import functools

import jax
import jax.numpy as jnp
from jax import lax
from jax.experimental import pallas as pl
from jax.experimental.pallas import tpu as pltpu
from jax.experimental.pallas import tpu_sc as plsc

NUM_CLASSES = 512
EMB_DIM = 512
N_SUPPORT = 16384
N_QUERY = 8192

_NC = 2
_NS = 16
_NW = _NC * _NS
_RPW = N_SUPPORT // _NW
_CHUNK = 64
_NCHUNKS = _RPW // _CHUNK
_NV = EMB_DIM // 16


def _sc_segment_sum_body(support_hbm, labels_hbm, nextlo_hbm,
                         dense_hbm, side_hbm,
                         rows_a, rows_b, flush_slots, zeros_v, labs_s,
                         nextlo_s, sem_a, sem_b, fsem):
    cid = lax.axis_index("c")
    sid = lax.axis_index("s")
    w = cid * _NS + sid
    row0 = w * _RPW

    pltpu.sync_copy(labels_hbm.at[pl.ds(row0, _RPW)],
                    labs_s.at[pl.ds(0, _RPW)])
    pltpu.sync_copy(nextlo_hbm, nextlo_s.at[pl.ds(0, _NW)])

    def _lab_at(i):
        return labs_s[pl.ds(i, 16)][0]

    for j in range(_NV):
        zeros_v[pl.ds(j * 16, 16)] = jnp.zeros((16,), jnp.float32)

    lo = _lab_at(0)
    nextlo = nextlo_s[pl.ds(w, 16)][0]

    def _zero_dense(c, _):
        pltpu.async_copy(zeros_v, dense_hbm.at[pl.ds(c * EMB_DIM, EMB_DIM)],
                         fsem)
        return 0

    @pl.when(w == 0)
    def _():
        lax.fori_loop(0, lo, _zero_dense, 0)

    def _flush(cls, acc, slot):
        base = (slot % 32) * EMB_DIM
        for j in range(_NV):
            flush_slots[pl.ds(base + j * 16, 16)] = acc[j]

        @pl.when(cls == lo)
        def _():
            pltpu.async_copy(flush_slots.at[pl.ds(base, EMB_DIM)],
                             side_hbm.at[pl.ds(2 * w * EMB_DIM, EMB_DIM)],
                             fsem)

        @pl.when(cls != lo)
        def _():
            pltpu.async_copy(flush_slots.at[pl.ds(base, EMB_DIM)],
                             dense_hbm.at[pl.ds(cls * EMB_DIM, EMB_DIM)],
                             fsem)

    zero16 = jnp.zeros((16,), jnp.float32)

    def _gather_start(k, buf, sem):
        pltpu.async_copy(
            support_hbm.at[pl.ds((row0 + k * _CHUNK) * EMB_DIM,
                                 _CHUNK * EMB_DIM)], buf, sem)

    def _gather_wait(buf, sem):
        pltpu.make_async_copy(
            support_hbm.at[pl.ds(0, _CHUNK * EMB_DIM)], buf, sem).wait()

    def _chunk_walk(buf, k, carry):
        def _row(r, carry):
            cur = carry[0]
            nf = carry[1]
            acc = carry[2:]
            lab = _lab_at(k * _CHUNK + r)
            changed = lab != cur

            @pl.when(changed)
            def _():
                _flush(cur, acc, nf)
                lax.fori_loop(cur + 1, lab, _zero_dense, 0)

            row = [buf[pl.ds(r * EMB_DIM + j * 16, 16)]
                   for j in range(_NV)]
            new_acc = [
                jnp.where(changed, row[j], acc[j] + row[j])
                for j in range(_NV)
            ]
            return (lab, jnp.where(changed, nf + (lab - cur), nf)) \
                + tuple(new_acc)

        return lax.fori_loop(0, _CHUNK, _row, carry)

    _gather_start(0, rows_a, sem_a)

    def _outer(t, carry):
        k0 = 2 * t
        _gather_wait(rows_a, sem_a)
        _gather_start(k0 + 1, rows_b, sem_b)
        carry = _chunk_walk(rows_a, k0, carry)
        _gather_wait(rows_b, sem_b)

        @pl.when(t < _NCHUNKS // 2 - 1)
        def _():
            _gather_start(k0 + 2, rows_a, sem_a)

        return _chunk_walk(rows_b, k0 + 1, carry)

    nf0 = jnp.where(w == 0, lo, 0)
    init = (lo, nf0) + tuple(zero16 for _ in range(_NV))
    final = lax.fori_loop(0, _NCHUNKS // 2, _outer, init)
    hi = final[0]
    nf = final[1]
    acc = final[2:]

    _flush(hi, acc, nf)

    @pl.when(hi == lo)
    def _():
        pltpu.async_copy(zeros_v,
                         side_hbm.at[pl.ds((2 * w + 1) * EMB_DIM, EMB_DIM)],
                         fsem)

    lax.fori_loop(hi, nextlo + 1, _zero_dense, 0)

    total = nf + 1 + jnp.where(hi == lo, 1, 0) + (nextlo + 1 - hi)

    def _drain(i, _):
        pltpu.make_async_copy(
            flush_slots.at[pl.ds(0, EMB_DIM)],
            dense_hbm.at[pl.ds(0, EMB_DIM)], fsem).wait()
        return 0

    lax.fori_loop(0, total, _drain, 0)


@functools.lru_cache(maxsize=1)
def _build_sc_segment_sum():
    return functools.partial(
        pl.kernel,
        out_type=(
            jax.ShapeDtypeStruct((NUM_CLASSES * EMB_DIM,), jnp.float32),
            jax.ShapeDtypeStruct((2 * _NW * EMB_DIM,), jnp.float32),
        ),
        mesh=plsc.VectorSubcoreMesh(core_axis_name="c", subcore_axis_name="s",
                                    num_cores=_NC, num_subcores=_NS),
        scratch_types=[
            pltpu.VMEM((_CHUNK * EMB_DIM,), jnp.float32),
            pltpu.VMEM((_CHUNK * EMB_DIM,), jnp.float32),
            pltpu.VMEM((32 * EMB_DIM,), jnp.float32),
            pltpu.VMEM((EMB_DIM,), jnp.float32),
            pltpu.VMEM((_RPW + 16,), jnp.int32),
            pltpu.VMEM((_NW + 16,), jnp.int32),
            pltpu.SemaphoreType.DMA,
            pltpu.SemaphoreType.DMA,
            pltpu.SemaphoreType.DMA,
        ],
    )(_sc_segment_sum_body)


_QBLK = 256
_LABCHUNK = 2048


def _tc_enc_body(q_ref, w_ref, b_ref, qe_ref):
    qe_ref[...] = lax.dot_general(
        q_ref[...], w_ref[...], (((1,), (1,)), ((), ())),
        preferred_element_type=jnp.float32) + b_ref[...]


_tc_enc = pl.pallas_call(
    _tc_enc_body,
    grid=(N_QUERY // _QBLK,),
    in_specs=[
        pl.BlockSpec((_QBLK, EMB_DIM), lambda i: (i, 0)),
        pl.BlockSpec((EMB_DIM, EMB_DIM), lambda i: (0, 0)),
        pl.BlockSpec((1, EMB_DIM), lambda i: (0, 0)),
    ],
    out_specs=pl.BlockSpec((_QBLK, EMB_DIM), lambda i: (i, 0)),
    out_shape=jax.ShapeDtypeStruct((N_QUERY, EMB_DIM), jnp.float32),
)


def _tc_main_body(qe_in_ref, w_ref, b_ref, dense_ref, side_ref, bidx_ref,
                  lab_ref, out_ref, protos_ref, p2_ref):
    i = pl.program_id(0)

    @pl.when(i == 0)
    def _():
        cls = lax.broadcasted_iota(jnp.int32, (NUM_CLASSES, 1), 0)
        clsf = cls.astype(jnp.float32)
        onehot = (clsf == bidx_ref[...]).astype(jnp.float32)
        sums = dense_ref[...] + lax.dot_general(
            onehot, side_ref[...], (((1,), (0,)), ((), ())),
            preferred_element_type=jnp.float32)
        acc = jnp.zeros((NUM_CLASSES, 1), jnp.float32)
        labs = lab_ref[...]
        for r in range(N_SUPPORT // _LABCHUNK):
            row = labs[r:r + 1, :]
            acc = acc + jnp.sum((row == cls).astype(jnp.float32), axis=1,
                                keepdims=True)
        inv = 1.0 / jnp.maximum(acc, 1.0)
        pm = lax.dot_general(sums, w_ref[...], (((1,), (1,)), ((), ())),
                             preferred_element_type=jnp.float32)
        protos = pm * inv + (acc * inv) * b_ref[...]
        protos_ref[...] = protos
        ones = jnp.ones((1, EMB_DIM), jnp.float32)
        p2_ref[...] = lax.dot_general(ones, protos * protos,
                                      (((1,), (1,)), ((), ())),
                                      preferred_element_type=jnp.float32)

    qe = qe_in_ref[...]
    protos = protos_ref[...]
    cross = lax.dot_general(qe, protos, (((1,), (1,)), ((), ())),
                            preferred_element_type=jnp.float32)
    q2 = jnp.sum(qe * qe, axis=1, keepdims=True)
    d2 = q2 + p2_ref[0:1, :] - 2.0 * cross
    out_ref[...] = -jnp.sqrt(jnp.maximum(d2, 1e-12))


_tc_main = pl.pallas_call(
    _tc_main_body,
    grid=(N_QUERY // _QBLK,),
    in_specs=[
        pl.BlockSpec((_QBLK, EMB_DIM), lambda i: (i, 0)),
        pl.BlockSpec((EMB_DIM, EMB_DIM), lambda i: (0, 0)),
        pl.BlockSpec((1, EMB_DIM), lambda i: (0, 0)),
        pl.BlockSpec((NUM_CLASSES, EMB_DIM), lambda i: (0, 0)),
        pl.BlockSpec((2 * _NW, EMB_DIM), lambda i: (0, 0)),
        pl.BlockSpec((1, 2 * _NW), lambda i: (0, 0)),
        pl.BlockSpec((N_SUPPORT // _LABCHUNK, _LABCHUNK), lambda i: (0, 0)),
    ],
    out_specs=pl.BlockSpec((_QBLK, NUM_CLASSES), lambda i: (i, 0)),
    out_shape=jax.ShapeDtypeStruct((N_QUERY, NUM_CLASSES), jnp.float32),
    scratch_shapes=[
        pltpu.VMEM((NUM_CLASSES, EMB_DIM), jnp.float32),
        pltpu.VMEM((1, NUM_CLASSES), jnp.float32),
    ],
)


def kernel(query_set, support_set, support_labels, W, b):
    labels = support_labels.astype(jnp.int32)
    nextlo = jnp.concatenate(
        [labels[_RPW::_RPW],
         jnp.full((1,), NUM_CLASSES, jnp.int32)])
    dense, side = _build_sc_segment_sum()(support_set.reshape(-1), labels,
                                          nextlo)
    dense = dense.reshape(NUM_CLASSES, EMB_DIM)
    side = side.reshape(2 * _NW, EMB_DIM)
    heads = labels[::_RPW]
    tails = labels[_RPW - 1::_RPW]
    bidx = jnp.stack([heads, tails], axis=1).reshape(1, 2 * _NW)
    bidx = bidx.astype(jnp.float32)
    b2 = b.reshape(1, EMB_DIM)
    lab2d = labels.reshape(N_SUPPORT // _LABCHUNK, _LABCHUNK)
    qe = _tc_enc(query_set, W, b2)
    return _tc_main(qe, W, b2, dense, side, bidx, lab2d)
